# baseline (speedup 1.0000x reference)
import functools

import jax
import jax.numpy as jnp
from jax import lax
from jax.experimental import pallas as pl
from jax.experimental.pallas import tpu as pltpu
from jax.experimental.pallas import tpu_sc as plsc

N = 10000
E = 320000
D = 128
RB = 8
H = 64
EPSILON = 0.125

NUM_WORKERS = 32
EPW = E // NUM_WORKERS
C = 40
NCH = EPW // C
ZROWS = 16
NPS = 624


_LOG2E = 1.4426950408889634


def _silu(x):
    return x / (1.0 + jnp.exp2(x * (-_LOG2E)))


def _pack_bf16_halves(x):
    k = x.shape[1] // 2
    lo_bits = jax.lax.bitcast_convert_type(x[:, :k], jnp.int32) + 0x8000
    hi_bits = jax.lax.bitcast_convert_type(x[:, k:], jnp.int32) + 0x8000
    lo16 = jax.lax.shift_right_logical(lo_bits, 16)
    hi16 = hi_bits & jnp.int32(-65536)
    return lo16 | hi16


def _h_body(nf_ref, w_ref, o_ref):
    o_ref[...] = jnp.dot(nf_ref[...],
                         w_ref[...] * (1.0 / jnp.sqrt(jnp.float32(D))),
                         preferred_element_type=jnp.float32)


def _mix_body(rbt_ref, w1_ref, w2_ref, w3_ref, w4_ref, o_ref):
    x = lax.dot_general(
        rbt_ref[...], w1_ref[...] * (1.0 / jnp.sqrt(jnp.float32(RB))),
        dimension_numbers=(((0,), (0,)), ((), ())),
        preferred_element_type=jnp.float32)
    x = _silu(x)
    x = jnp.dot(x, w2_ref[...] * (1.0 / jnp.sqrt(jnp.float32(H))),
                preferred_element_type=jnp.float32)
    x = _silu(x)
    x = jnp.dot(x, w3_ref[...] * (1.0 / jnp.sqrt(jnp.float32(H))),
                preferred_element_type=jnp.float32)
    x = _silu(x)
    mf = jnp.dot(
        x, w4_ref[...] * (1.0 / (jnp.sqrt(jnp.float32(H)) * jnp.float32(D))),
        preferred_element_type=jnp.float32)
    o_ref[...] = _pack_bf16_halves(mf)


def _out_body(acca_ref, accb_ref, w_ref, o_ref):
    s = acca_ref[0] + acca_ref[1] + accb_ref[0] + accb_ref[1]
    o_ref[...] = jnp.dot(
        s, w_ref[...] * (EPSILON / jnp.sqrt(jnp.float32(D))),
        preferred_element_type=jnp.float32)


def _make_sc_body(epw):
    nch = epw // C
    assert epw % C == 0 and nch >= 8
    tail = (nch - 4) % 4
    nquad = (nch - 4 - tail) // 4

    def _sc_body(h_hbm, mix_hbm, snd_hbm, rcv_hbm, out_hbm,
                 rows0, rows1, rows2, rows3, mix0, mix1, mix2, mix3,
                 si0, si1, si2, si3, ri0, ri1, ri2, ri3, acc,
                 gs0, gs1, gs2, gs3, ms0, ms1, ms2, ms3,
                 ss0, ss1, ss2, ss3, ts0, ts1, ts2, ts3, rs0, rs1, rs2, rs3):
        c = lax.axis_index("c")
        s = lax.axis_index("s")
        w = c * 16 + s

        rowb = (rows0, rows1, rows2, rows3)
        mixb = (mix0, mix1, mix2, mix3)
        sidxb = (si0, si1, si2, si3)
        ridxb = (ri0, ri1, ri2, ri3)
        gsem = (gs0, gs1, gs2, gs3)
        msem = (ms0, ms1, ms2, ms3)
        ssem = (ss0, ss1, ss2, ss3)
        tsem = (ts0, ts1, ts2, ts3)
        rsem = (rs0, rs1, rs2, rs3)

        zero16 = jnp.zeros((16,), jnp.float32)
        ntr = jnp.where(s == 15, (N - 15 * NPS) // ZROWS, NPS // ZROWS)

        def zfill(i, carry):
            for j in range(D // 16):
                rows0[i, pl.ds(j * 16, 16)] = zero16
            return carry

        lax.fori_loop(0, ZROWS, zfill, 0)

        def zcopy(k, carry):
            pltpu.sync_copy(rows0.at[pl.ds(0, ZROWS)],
                            acc.at[pl.ds(s * NPS + k * ZROWS, ZROWS)])
            return carry

        lax.fori_loop(0, ntr, zcopy, 0)
        plsc.subcore_barrier()

        base0 = w * epw

        def start_sidx(k, b):
            pltpu.async_copy(snd_hbm.at[w, k], sidxb[b], tsem[b])

        def wait_sidx(b):
            pltpu.make_async_copy(snd_hbm.at[0, 0], sidxb[b], tsem[b]).wait()

        def start_ridx(k, b):
            pltpu.async_copy(rcv_hbm.at[w, k], ridxb[b], rsem[b])

        def wait_ridx(b):
            pltpu.make_async_copy(rcv_hbm.at[0, 0], ridxb[b], rsem[b]).wait()

        def start_gm(k, b):
            pltpu.async_copy(h_hbm.at[sidxb[b]], rowb[b], gsem[b])
            pltpu.async_copy(mix_hbm.at[pl.ds(base0 + k * C, C)],
                             mixb[b], msem[b])

        def wait_gm(b):
            pltpu.make_async_copy(h_hbm.at[sidxb[b]], rowb[b], gsem[b]).wait()
            pltpu.make_async_copy(mix_hbm.at[pl.ds(0, C)],
                                  mixb[b], msem[b]).wait()

        def start_scatter(b):
            pltpu.async_copy(rowb[b], acc.at[ridxb[b]], ssem[b], add=True)

        def wait_scatter(b):
            pltpu.make_async_copy(rowb[b], acc.at[ridxb[b]], ssem[b]).wait()

        himask = jnp.int32(-65536)

        def mul(b):
            rows, mixv = rowb[b], mixb[b]
            bc = lambda v: jax.lax.bitcast_convert_type(v, jnp.float32)

            def mrow(i, cc):
                for r in range(2):
                    row = 2 * i + r
                    for g in range(D // 32):
                        sl = pl.ds(g * 16, 16)
                        mv = mixv[row, sl]
                        slo = pl.ds(g * 16, 16)
                        shi = pl.ds(D // 2 + g * 16, 16)
                        rows[row, slo] = rows[row, slo] * bc(mv << 16)
                        rows[row, shi] = rows[row, shi] * bc(mv & himask)
                return cc

            lax.fori_loop(0, C // 2, mrow, 0)

        for x in range(4):
            start_sidx(x, x)
        start_ridx(0, 0)
        start_ridx(1, 1)
        wait_sidx(0)
        start_gm(0, 0)
        wait_sidx(1)
        start_gm(1, 1)

        for r in range(4):
            bn = (r + 2) % 4
            if r >= 2:
                wait_scatter(bn)
            start_ridx(r + 2, bn)
            wait_sidx(bn)
            start_gm(r + 2, bn)
            wait_gm(r)
            start_sidx(r + 4, r)
            mul(r)
            wait_ridx(r)
            start_scatter(r)

        def quad(i, cc):
            for r in range(4):
                k = 4 * i + r
                bn = (r + 2) % 4
                wait_scatter(bn)

                @pl.when(k + 2 < nch)
                def _():
                    start_ridx(k + 2, bn)
                    wait_sidx(bn)
                    start_gm(k + 2, bn)

                wait_gm(r)

                @pl.when(k + 4 < nch)
                def _():
                    start_sidx(k + 4, r)

                mul(r)
                wait_ridx(r)
                start_scatter(r)
            return cc

        lax.fori_loop(1, 1 + nquad, quad, 0)

        for k in range(nch - tail, nch):
            b = k % 4
            wait_scatter((k + 2) % 4)
            wait_gm(b)
            mul(b)
            wait_ridx(b)
            start_scatter(b)
        wait_scatter((nch - 2) % 4)
        wait_scatter((nch - 1) % 4)
        plsc.subcore_barrier()

        def wout(k, carry):
            sl = pl.ds(s * NPS + k * ZROWS, ZROWS)
            pltpu.sync_copy(acc.at[sl], out_hbm.at[c, sl])
            return carry

        lax.fori_loop(0, ntr, wout, 0)

    return _sc_body


def _make_sc_call(epw):
    return functools.partial(
        pl.kernel,
        mesh=plsc.VectorSubcoreMesh(core_axis_name="c", subcore_axis_name="s"),
        out_type=jax.ShapeDtypeStruct((2, N, D), jnp.float32),
        scratch_types=(
            [pltpu.VMEM((C, D), jnp.float32)] * 4
            + [pltpu.VMEM((C, D // 2), jnp.int32)] * 4
            + [pltpu.VMEM((C,), jnp.int32)] * 8
            + [pltpu.VMEM_SHARED((N, D), jnp.float32)]
            + [pltpu.SemaphoreType.DMA] * 20
        ),
    )(_make_sc_body(epw))


def kernel(vectors, node_feats, radial_embedding, senders, receivers,
           W_up, W_mlp1, W_mlp2, W_mlp3, W_mlp4, W_down):
    f32 = jnp.float32

    h = pl.pallas_call(
        _h_body,
        grid=(10,),
        in_specs=[
            pl.BlockSpec((N // 10, D), lambda i: (i, 0)),
            pl.BlockSpec((D, D), lambda i: (0, 0)),
        ],
        out_specs=pl.BlockSpec((N // 10, D), lambda i: (i, 0)),
        out_shape=jax.ShapeDtypeStruct((N, D), f32),
    )(node_feats, W_up)

    E2 = E // 2
    EPW2 = E2 // NUM_WORKERS
    EB = 6400
    rbt = radial_embedding.T

    def mix_half(off_blocks):
        return pl.pallas_call(
            _mix_body,
            grid=(E2 // EB,),
            in_specs=[
                pl.BlockSpec((RB, EB), lambda i, o=off_blocks: (0, i + o)),
                pl.BlockSpec((RB, H), lambda i: (0, 0)),
                pl.BlockSpec((H, H), lambda i: (0, 0)),
                pl.BlockSpec((H, H), lambda i: (0, 0)),
                pl.BlockSpec((H, D), lambda i: (0, 0)),
            ],
            out_specs=pl.BlockSpec((EB, D // 2), lambda i: (i, 0)),
            out_shape=jax.ShapeDtypeStruct((E2, D // 2), jnp.int32),
        )(rbt, W_mlp1, W_mlp2, W_mlp3, W_mlp4)

    sc_call = _make_sc_call(EPW2)
    snd = senders.reshape(2, NUM_WORKERS, EPW2 // C, C)
    rcv = receivers.reshape(2, NUM_WORKERS, EPW2 // C, C)

    mix_a = mix_half(0)
    acc_a = sc_call(h, mix_a, snd[0], rcv[0])
    mix_b = mix_half(E2 // EB)
    acc_b = sc_call(h, mix_b, snd[1], rcv[1])

    out = pl.pallas_call(
        _out_body,
        grid=(10,),
        in_specs=[
            pl.BlockSpec((2, N // 10, D), lambda i: (0, i, 0)),
            pl.BlockSpec((2, N // 10, D), lambda i: (0, i, 0)),
            pl.BlockSpec((D, D), lambda i: (0, 0)),
        ],
        out_specs=pl.BlockSpec((N // 10, D), lambda i: (i, 0)),
        out_shape=jax.ShapeDtypeStruct((N, D), f32),
    )(acc_a, acc_b, W_down)
    return out


# speedup vs baseline: 1.1827x; 1.1827x over previous
import functools

import jax
import jax.numpy as jnp
from jax import lax
from jax.experimental import pallas as pl
from jax.experimental.pallas import tpu as pltpu
from jax.experimental.pallas import tpu_sc as plsc

N = 10000
E = 320000
D = 128
RB = 8
H = 64
EPSILON = 0.125

NUM_WORKERS = 32
EPW = E // NUM_WORKERS
C = 40
NCH = EPW // C
ZROWS = 16
NPS = 624


_LOG2E = 1.4426950408889634


def _silu(x):
    return x / (1.0 + jnp.exp2(x * (-_LOG2E)))


def _pack_bf16_halves(x):
    k = x.shape[1] // 2
    lo_bits = jax.lax.bitcast_convert_type(x[:, :k], jnp.int32) + 0x8000
    hi_bits = jax.lax.bitcast_convert_type(x[:, k:], jnp.int32) + 0x8000
    lo16 = jax.lax.shift_right_logical(lo_bits, 16)
    hi16 = hi_bits & jnp.int32(-65536)
    return lo16 | hi16


def _h_body(nf_ref, w_ref, o_ref):
    o_ref[...] = jnp.dot(nf_ref[...],
                         w_ref[...] * (1.0 / jnp.sqrt(jnp.float32(D))),
                         preferred_element_type=jnp.float32)


def _mix_body(rbt_ref, w1_ref, w2_ref, w3_ref, w4_ref, o_ref):
    x = lax.dot_general(
        rbt_ref[...], w1_ref[...] * (1.0 / jnp.sqrt(jnp.float32(RB))),
        dimension_numbers=(((0,), (0,)), ((), ())),
        preferred_element_type=jnp.float32)
    x = _silu(x)
    x = jnp.dot(x, w2_ref[...] * (1.0 / jnp.sqrt(jnp.float32(H))),
                preferred_element_type=jnp.float32)
    x = _silu(x)
    x = jnp.dot(x, w3_ref[...] * (1.0 / jnp.sqrt(jnp.float32(H))),
                preferred_element_type=jnp.float32)
    x = _silu(x)
    mf = jnp.dot(
        x, w4_ref[...] * (1.0 / (jnp.sqrt(jnp.float32(H)) * jnp.float32(D))),
        preferred_element_type=jnp.float32)
    o_ref[...] = _pack_bf16_halves(mf)


def _out_body(acca_ref, accb_ref, w_ref, o_ref):
    s = acca_ref[0] + acca_ref[1] + accb_ref[0] + accb_ref[1]
    o_ref[...] = jnp.dot(
        s, w_ref[...] * (EPSILON / jnp.sqrt(jnp.float32(D))),
        preferred_element_type=jnp.float32)


def _make_sc_body(epw, edge0):
    nch = epw // C
    assert epw % C == 0 and nch >= 8
    tail = (nch - 4) % 4
    nquad = (nch - 4 - tail) // 4

    def _sc_body(h_hbm, mix_hbm, snd_hbm, rcv_hbm, out_hbm,
                 rows0, rows1, rows2, rows3, mix0, mix1, mix2, mix3,
                 si0, si1, si2, si3, ri0, ri1, ri2, ri3, acc,
                 gs0, gs1, gs2, gs3, ms0, ms1, ms2, ms3,
                 ss0, ss1, ss2, ss3, ts0, ts1, ts2, ts3, rs0, rs1, rs2, rs3):
        c = lax.axis_index("c")
        s = lax.axis_index("s")
        w = c * 16 + s

        rowb = (rows0, rows1, rows2, rows3)
        mixb = (mix0, mix1, mix2, mix3)
        sidxb = (si0, si1, si2, si3)
        ridxb = (ri0, ri1, ri2, ri3)
        gsem = (gs0, gs1, gs2, gs3)
        msem = (ms0, ms1, ms2, ms3)
        ssem = (ss0, ss1, ss2, ss3)
        tsem = (ts0, ts1, ts2, ts3)
        rsem = (rs0, rs1, rs2, rs3)

        zero16 = jnp.zeros((16,), jnp.float32)
        ntr = jnp.where(s == 15, (N - 15 * NPS) // ZROWS, NPS // ZROWS)

        def zfill(i, carry):
            for j in range(D // 16):
                rows0[i, pl.ds(j * 16, 16)] = zero16
            return carry

        lax.fori_loop(0, ZROWS, zfill, 0)

        def zcopy(k, carry):
            pltpu.async_copy(rows0.at[pl.ds(0, ZROWS)],
                             acc.at[pl.ds(s * NPS + k * ZROWS, ZROWS)],
                             gs0)
            return carry

        lax.fori_loop(0, ntr, zcopy, 0)

        def zdrain(k, carry):
            pltpu.make_async_copy(
                rows0.at[pl.ds(0, ZROWS)],
                acc.at[pl.ds(s * NPS, ZROWS)], gs0).wait()
            return carry

        lax.fori_loop(0, ntr, zdrain, 0)
        plsc.subcore_barrier()

        base0 = w * epw
        ibase0 = edge0 + w * epw

        def start_sidx(k, b):
            pltpu.async_copy(snd_hbm.at[pl.ds(ibase0 + k * C, C)],
                             sidxb[b], tsem[b])

        def wait_sidx(b):
            pltpu.make_async_copy(snd_hbm.at[pl.ds(0, C)],
                                  sidxb[b], tsem[b]).wait()

        def start_ridx(k, b):
            pltpu.async_copy(rcv_hbm.at[pl.ds(ibase0 + k * C, C)],
                             ridxb[b], rsem[b])

        def wait_ridx(b):
            pltpu.make_async_copy(rcv_hbm.at[pl.ds(0, C)],
                                  ridxb[b], rsem[b]).wait()

        def start_gm(k, b):
            pltpu.async_copy(h_hbm.at[sidxb[b]], rowb[b], gsem[b])
            pltpu.async_copy(mix_hbm.at[pl.ds(base0 + k * C, C)],
                             mixb[b], msem[b])

        def wait_gm(b):
            pltpu.make_async_copy(h_hbm.at[sidxb[b]], rowb[b], gsem[b]).wait()
            pltpu.make_async_copy(mix_hbm.at[pl.ds(0, C)],
                                  mixb[b], msem[b]).wait()

        def start_scatter(b):
            pltpu.async_copy(rowb[b], acc.at[ridxb[b]], ssem[b], add=True)

        def wait_scatter(b):
            pltpu.make_async_copy(rowb[b], acc.at[ridxb[b]], ssem[b]).wait()

        himask = jnp.int32(-65536)

        def mul(b):
            rows, mixv = rowb[b], mixb[b]
            bc = lambda v: jax.lax.bitcast_convert_type(v, jnp.float32)

            def mrow(i, cc):
                for r in range(2):
                    row = 2 * i + r
                    for g in range(D // 32):
                        sl = pl.ds(g * 16, 16)
                        mv = mixv[row, sl]
                        slo = pl.ds(g * 16, 16)
                        shi = pl.ds(D // 2 + g * 16, 16)
                        rows[row, slo] = rows[row, slo] * bc(mv << 16)
                        rows[row, shi] = rows[row, shi] * bc(mv & himask)
                return cc

            lax.fori_loop(0, C // 2, mrow, 0)

        for x in range(4):
            start_sidx(x, x)
        start_ridx(0, 0)
        start_ridx(1, 1)
        wait_sidx(0)
        start_gm(0, 0)
        wait_sidx(1)
        start_gm(1, 1)

        for r in range(4):
            bn = (r + 2) % 4
            if r >= 2:
                wait_scatter(bn)
            start_ridx(r + 2, bn)
            wait_sidx(bn)
            start_gm(r + 2, bn)
            wait_gm(r)
            start_sidx(r + 4, r)
            mul(r)
            wait_ridx(r)
            start_scatter(r)

        def quad(i, cc):
            for r in range(4):
                k = 4 * i + r
                bn = (r + 2) % 4
                wait_scatter(bn)

                @pl.when(k + 2 < nch)
                def _():
                    start_ridx(k + 2, bn)
                    wait_sidx(bn)
                    start_gm(k + 2, bn)

                wait_gm(r)

                @pl.when(k + 4 < nch)
                def _():
                    start_sidx(k + 4, r)

                mul(r)
                wait_ridx(r)
                start_scatter(r)
            return cc

        lax.fori_loop(1, 1 + nquad, quad, 0)

        for k in range(nch - tail, nch):
            b = k % 4
            wait_scatter((k + 2) % 4)
            wait_gm(b)
            mul(b)
            wait_ridx(b)
            start_scatter(b)
        wait_scatter((nch - 2) % 4)
        wait_scatter((nch - 1) % 4)
        plsc.subcore_barrier()

        def wout(k, carry):
            sl = pl.ds(s * NPS + k * ZROWS, ZROWS)
            pltpu.async_copy(acc.at[sl], out_hbm.at[c, sl], gs0)
            return carry

        lax.fori_loop(0, ntr, wout, 0)

        def wdrain(k, carry):
            sl = pl.ds(s * NPS, ZROWS)
            pltpu.make_async_copy(acc.at[sl], out_hbm.at[c, sl], gs0).wait()
            return carry

        lax.fori_loop(0, ntr, wdrain, 0)

    return _sc_body


def _make_sc_call(epw, edge0):
    return functools.partial(
        pl.kernel,
        mesh=plsc.VectorSubcoreMesh(core_axis_name="c", subcore_axis_name="s"),
        out_type=jax.ShapeDtypeStruct((2, N, D), jnp.float32),
        scratch_types=(
            [pltpu.VMEM((C, D), jnp.float32)] * 4
            + [pltpu.VMEM((C, D // 2), jnp.int32)] * 4
            + [pltpu.VMEM((C,), jnp.int32)] * 8
            + [pltpu.VMEM_SHARED((N, D), jnp.float32)]
            + [pltpu.SemaphoreType.DMA] * 20
        ),
    )(_make_sc_body(epw, edge0))


def kernel(vectors, node_feats, radial_embedding, senders, receivers,
           W_up, W_mlp1, W_mlp2, W_mlp3, W_mlp4, W_down):
    f32 = jnp.float32

    h = pl.pallas_call(
        _h_body,
        grid=(10,),
        in_specs=[
            pl.BlockSpec((N // 10, D), lambda i: (i, 0)),
            pl.BlockSpec((D, D), lambda i: (0, 0)),
        ],
        out_specs=pl.BlockSpec((N // 10, D), lambda i: (i, 0)),
        out_shape=jax.ShapeDtypeStruct((N, D), f32),
    )(node_feats, W_up)

    E2 = E // 2
    EPW2 = E2 // NUM_WORKERS
    EB = 6400
    rbt = radial_embedding.T

    def mix_half(off_blocks):
        return pl.pallas_call(
            _mix_body,
            grid=(E2 // EB,),
            in_specs=[
                pl.BlockSpec((RB, EB), lambda i, o=off_blocks: (0, i + o)),
                pl.BlockSpec((RB, H), lambda i: (0, 0)),
                pl.BlockSpec((H, H), lambda i: (0, 0)),
                pl.BlockSpec((H, H), lambda i: (0, 0)),
                pl.BlockSpec((H, D), lambda i: (0, 0)),
            ],
            out_specs=pl.BlockSpec((EB, D // 2), lambda i: (i, 0)),
            out_shape=jax.ShapeDtypeStruct((E2, D // 2), jnp.int32),
        )(rbt, W_mlp1, W_mlp2, W_mlp3, W_mlp4)

    mix_a = mix_half(0)
    acc_a = _make_sc_call(EPW2, 0)(h, mix_a, senders, receivers)
    mix_b = mix_half(E2 // EB)
    acc_b = _make_sc_call(EPW2, E2)(h, mix_b, senders, receivers)

    out = pl.pallas_call(
        _out_body,
        grid=(10,),
        in_specs=[
            pl.BlockSpec((2, N // 10, D), lambda i: (0, i, 0)),
            pl.BlockSpec((2, N // 10, D), lambda i: (0, i, 0)),
            pl.BlockSpec((D, D), lambda i: (0, 0)),
        ],
        out_specs=pl.BlockSpec((N // 10, D), lambda i: (i, 0)),
        out_shape=jax.ShapeDtypeStruct((N, D), f32),
    )(acc_a, acc_b, W_down)
    return out


# speedup vs baseline: 1.1916x; 1.0075x over previous
import functools

import jax
import jax.numpy as jnp
from jax import lax
from jax.experimental import pallas as pl
from jax.experimental.pallas import tpu as pltpu
from jax.experimental.pallas import tpu_sc as plsc

N = 10000
E = 320000
D = 128
RB = 8
H = 64
EPSILON = 0.125

NUM_WORKERS = 32
EPW = E // NUM_WORKERS
C = 40
NCH = EPW // C
ZROWS = 16
NPS = 624


_LOG2E = 1.4426950408889634


def _silu(x):
    return x / (1.0 + jnp.exp2(x * (-_LOG2E)))


def _h_body(nf_ref, w_ref, o_ref):
    o_ref[...] = jnp.dot(nf_ref[...],
                         w_ref[...] * (1.0 / jnp.sqrt(jnp.float32(D))),
                         preferred_element_type=jnp.float32)


def _mix_body(rbt_ref, w1_ref, w2_ref, w3_ref, w4_ref, o_ref):
    x = lax.dot_general(
        rbt_ref[...], w1_ref[...] * (1.0 / jnp.sqrt(jnp.float32(RB))),
        dimension_numbers=(((0,), (0,)), ((), ())),
        preferred_element_type=jnp.float32)
    x = _silu(x)
    x = jnp.dot(x, w2_ref[...] * (1.0 / jnp.sqrt(jnp.float32(H))),
                preferred_element_type=jnp.float32)
    x = _silu(x)
    x = jnp.dot(x, w3_ref[...] * (1.0 / jnp.sqrt(jnp.float32(H))),
                preferred_element_type=jnp.float32)
    x = _silu(x)
    o_ref[...] = jnp.dot(
        x, w4_ref[...] * (1.0 / (jnp.sqrt(jnp.float32(H)) * jnp.float32(D))),
        preferred_element_type=jnp.float32)


def _out_body(acca_ref, accb_ref, w_ref, o_ref):
    s = acca_ref[0] + acca_ref[1] + accb_ref[0] + accb_ref[1]
    o_ref[...] = jnp.dot(
        s, w_ref[...] * (EPSILON / jnp.sqrt(jnp.float32(D))),
        preferred_element_type=jnp.float32)


def _make_sc_body(epw, edge0):
    nch = epw // C
    assert epw % C == 0 and nch >= 8
    tail = (nch - 4) % 4
    nquad = (nch - 4 - tail) // 4

    def _sc_body(h_hbm, mix_hbm, snd_hbm, rcv_hbm, out_hbm,
                 rows0, rows1, rows2, rows3, mix0, mix1, mix2, mix3,
                 si0, si1, si2, si3, ri0, ri1, ri2, ri3, acc,
                 gs0, gs1, gs2, gs3, ms0, ms1, ms2, ms3,
                 ss0, ss1, ss2, ss3, ts0, ts1, ts2, ts3, rs0, rs1, rs2, rs3):
        c = lax.axis_index("c")
        s = lax.axis_index("s")
        w = c * 16 + s

        rowb = (rows0, rows1, rows2, rows3)
        mixb = (mix0, mix1, mix2, mix3)
        sidxb = (si0, si1, si2, si3)
        ridxb = (ri0, ri1, ri2, ri3)
        gsem = (gs0, gs1, gs2, gs3)
        msem = (ms0, ms1, ms2, ms3)
        ssem = (ss0, ss1, ss2, ss3)
        tsem = (ts0, ts1, ts2, ts3)
        rsem = (rs0, rs1, rs2, rs3)

        zero16 = jnp.zeros((16,), jnp.float32)
        ntr = jnp.where(s == 15, (N - 15 * NPS) // ZROWS, NPS // ZROWS)

        def zfill(i, carry):
            for j in range(D // 16):
                rows0[i, pl.ds(j * 16, 16)] = zero16
            return carry

        lax.fori_loop(0, ZROWS, zfill, 0)

        def zcopy(k, carry):
            pltpu.async_copy(rows0.at[pl.ds(0, ZROWS)],
                             acc.at[pl.ds(s * NPS + k * ZROWS, ZROWS)],
                             gs0)
            return carry

        lax.fori_loop(0, ntr, zcopy, 0)

        def zdrain(k, carry):
            pltpu.make_async_copy(
                rows0.at[pl.ds(0, ZROWS)],
                acc.at[pl.ds(s * NPS, ZROWS)], gs0).wait()
            return carry

        lax.fori_loop(0, ntr, zdrain, 0)
        plsc.subcore_barrier()

        base0 = w * epw
        ibase0 = edge0 + w * epw

        def start_sidx(k, b):
            pltpu.async_copy(snd_hbm.at[pl.ds(ibase0 + k * C, C)],
                             sidxb[b], tsem[b])

        def wait_sidx(b):
            pltpu.make_async_copy(snd_hbm.at[pl.ds(0, C)],
                                  sidxb[b], tsem[b]).wait()

        def start_ridx(k, b):
            pltpu.async_copy(rcv_hbm.at[pl.ds(ibase0 + k * C, C)],
                             ridxb[b], rsem[b])

        def wait_ridx(b):
            pltpu.make_async_copy(rcv_hbm.at[pl.ds(0, C)],
                                  ridxb[b], rsem[b]).wait()

        def start_gm(k, b):
            pltpu.async_copy(h_hbm.at[sidxb[b]], rowb[b], gsem[b])
            pltpu.async_copy(mix_hbm.at[pl.ds(base0 + k * C, C)],
                             mixb[b], msem[b])

        def wait_gm(b):
            pltpu.make_async_copy(h_hbm.at[sidxb[b]], rowb[b], gsem[b]).wait()
            pltpu.make_async_copy(mix_hbm.at[pl.ds(0, C)],
                                  mixb[b], msem[b]).wait()

        def start_scatter(b):
            pltpu.async_copy(rowb[b], acc.at[ridxb[b]], ssem[b], add=True)

        def wait_scatter(b):
            pltpu.make_async_copy(rowb[b], acc.at[ridxb[b]], ssem[b]).wait()

        def mul(b):
            rows, mixv = rowb[b], mixb[b]

            def mrow(i, cc):
                for r in range(2):
                    row = 2 * i + r
                    for j in range(D // 16):
                        sl = pl.ds(j * 16, 16)
                        rows[row, sl] = rows[row, sl] * mixv[row, sl]
                return cc

            lax.fori_loop(0, C // 2, mrow, 0)

        for x in range(4):
            start_sidx(x, x)
        start_ridx(0, 0)
        start_ridx(1, 1)
        wait_sidx(0)
        start_gm(0, 0)
        wait_sidx(1)
        start_gm(1, 1)

        for r in range(4):
            bn = (r + 2) % 4
            if r >= 2:
                wait_scatter(bn)
            start_ridx(r + 2, bn)
            wait_sidx(bn)
            start_gm(r + 2, bn)
            wait_gm(r)
            start_sidx(r + 4, r)
            mul(r)
            wait_ridx(r)
            start_scatter(r)

        def quad(i, cc):
            for r in range(4):
                k = 4 * i + r
                bn = (r + 2) % 4
                wait_scatter(bn)

                @pl.when(k + 2 < nch)
                def _():
                    start_ridx(k + 2, bn)
                    wait_sidx(bn)
                    start_gm(k + 2, bn)

                wait_gm(r)

                @pl.when(k + 4 < nch)
                def _():
                    start_sidx(k + 4, r)

                mul(r)
                wait_ridx(r)
                start_scatter(r)
            return cc

        lax.fori_loop(1, 1 + nquad, quad, 0)

        for k in range(nch - tail, nch):
            b = k % 4
            wait_scatter((k + 2) % 4)
            wait_gm(b)
            mul(b)
            wait_ridx(b)
            start_scatter(b)
        wait_scatter((nch - 2) % 4)
        wait_scatter((nch - 1) % 4)
        plsc.subcore_barrier()

        def wout(k, carry):
            sl = pl.ds(s * NPS + k * ZROWS, ZROWS)
            pltpu.async_copy(acc.at[sl], out_hbm.at[c, sl], gs0)
            return carry

        lax.fori_loop(0, ntr, wout, 0)

        def wdrain(k, carry):
            sl = pl.ds(s * NPS, ZROWS)
            pltpu.make_async_copy(acc.at[sl], out_hbm.at[c, sl], gs0).wait()
            return carry

        lax.fori_loop(0, ntr, wdrain, 0)

    return _sc_body


def _make_sc_call(epw, edge0):
    return functools.partial(
        pl.kernel,
        mesh=plsc.VectorSubcoreMesh(core_axis_name="c", subcore_axis_name="s"),
        out_type=jax.ShapeDtypeStruct((2, N, D), jnp.float32),
        scratch_types=(
            [pltpu.VMEM((C, D), jnp.float32)] * 8
            + [pltpu.VMEM((C,), jnp.int32)] * 8
            + [pltpu.VMEM_SHARED((N, D), jnp.float32)]
            + [pltpu.SemaphoreType.DMA] * 20
        ),
    )(_make_sc_body(epw, edge0))


def kernel(vectors, node_feats, radial_embedding, senders, receivers,
           W_up, W_mlp1, W_mlp2, W_mlp3, W_mlp4, W_down):
    f32 = jnp.float32

    h = pl.pallas_call(
        _h_body,
        grid=(10,),
        in_specs=[
            pl.BlockSpec((N // 10, D), lambda i: (i, 0)),
            pl.BlockSpec((D, D), lambda i: (0, 0)),
        ],
        out_specs=pl.BlockSpec((N // 10, D), lambda i: (i, 0)),
        out_shape=jax.ShapeDtypeStruct((N, D), f32),
    )(node_feats, W_up)

    E2 = E // 2
    EPW2 = E2 // NUM_WORKERS
    EB = 6400
    rbt = radial_embedding.T

    def mix_half(off_blocks):
        return pl.pallas_call(
            _mix_body,
            grid=(E2 // EB,),
            in_specs=[
                pl.BlockSpec((RB, EB), lambda i, o=off_blocks: (0, i + o)),
                pl.BlockSpec((RB, H), lambda i: (0, 0)),
                pl.BlockSpec((H, H), lambda i: (0, 0)),
                pl.BlockSpec((H, H), lambda i: (0, 0)),
                pl.BlockSpec((H, D), lambda i: (0, 0)),
            ],
            out_specs=pl.BlockSpec((EB, D), lambda i: (i, 0)),
            out_shape=jax.ShapeDtypeStruct((E2, D), f32),
        )(rbt, W_mlp1, W_mlp2, W_mlp3, W_mlp4)

    mix_a = mix_half(0)
    acc_a = _make_sc_call(EPW2, 0)(h, mix_a, senders, receivers)
    mix_b = mix_half(E2 // EB)
    acc_b = _make_sc_call(EPW2, E2)(h, mix_b, senders, receivers)

    out = pl.pallas_call(
        _out_body,
        grid=(10,),
        in_specs=[
            pl.BlockSpec((2, N // 10, D), lambda i: (0, i, 0)),
            pl.BlockSpec((2, N // 10, D), lambda i: (0, i, 0)),
            pl.BlockSpec((D, D), lambda i: (0, 0)),
        ],
        out_specs=pl.BlockSpec((N // 10, D), lambda i: (i, 0)),
        out_shape=jax.ShapeDtypeStruct((N, D), f32),
    )(acc_a, acc_b, W_down)
    return out
